# Initial kernel scaffold; baseline (speedup 1.0000x reference)
import jax
import jax.numpy as jnp
from jax.experimental import pallas as pl


def kernel(x, edge_index, batch, node_emb, W_struct, b_struct, W_emb, b_emb, W_comb, b_comb, W_tok, b_tok, W_layer, b_layer, W_h1, b_h1, W_h2, b_h2):
    raise NotImplementedError("write your pallas kernel here")


# speedup vs baseline: 43.1098x; 43.1098x over previous
import functools

import jax
import jax.numpy as jnp
from jax import lax
from jax.experimental import pallas as pl
from jax.experimental.pallas import tpu as pltpu
from jax.experimental.pallas import tpu_sc as plsc

N_NODES = 10000
N_EDGES = 320000
B_GRAPHS = 64
CHANNELS = 128

NC = 2
NS = 16
HALF = 5120
N_PAD = NC * HALF
NPT = HALF // NS
EPT = N_EDGES // NS

MIN_R = 0.2
MAX_R = 1.0
MIN_HG = 0.05
SIZE_MIX = 0.3
SIZE_TEMP = 3.0
HEAD_MIX = 0.3
HEAD_TEMP = 2.0
NUM_LAYERS = 12
NUM_HEADS = 12


def _sc_stats_body(src_hbm, batch_hbm, out_hbm,
                   e_v, hist_v, part_v, b_v, st_v, stred_v,
                   hist_sh, st_sh):
    c = lax.axis_index("c")
    s = lax.axis_index("s")
    lo = c * HALF

    zf = jnp.zeros((16,), jnp.float32)
    ones16 = jnp.ones((16,), jnp.float32)

    def zbody(i, carry):
        hist_v[pl.ds(i * 16, 16)] = zf
        return carry
    lax.fori_loop(0, HALF // 16, zbody, 0, unroll=4)

    pltpu.sync_copy(src_hbm.at[pl.ds(s * EPT, EPT)], e_v)

    def hbody(i, carry):
        idx = e_v[pl.ds(i * 16, 16)] - lo
        m = (idx >= 0) & (idx < HALF)
        plsc.addupdate_scatter(hist_v, [idx], ones16, mask=m)
        return carry
    lax.fori_loop(0, EPT // 16, hbody, 0, unroll=4)

    pltpu.sync_copy(hist_v, hist_sh.at[pl.ds(s * HALF, HALF)])
    plsc.subcore_barrier()

    for l in range(NS):
        pltpu.sync_copy(hist_sh.at[pl.ds(l * HALF + s * NPT, NPT)],
                        part_v.at[pl.ds(l * NPT, NPT)])

    def zsbody(i, carry):
        st_v[pl.ds(i * 16, 16)] = zf
        return carry
    lax.fori_loop(0, 512 // 16, zsbody, 0)

    pltpu.sync_copy(batch_hbm.at[pl.ds(lo + s * NPT, NPT)], b_v)

    nbase = lo + s * NPT
    lane = lax.iota(jnp.int32, 16)

    def sbody(j, carry):
        d = part_v[pl.ds(j * 16, 16)]
        for l in range(1, NS):
            d = d + part_v[pl.ds(l * NPT + j * 16, 16)]
        g = b_v[pl.ds(j * 16, 16)]
        gm = (nbase + j * 16 + lane) < N_NODES
        base = g * 8
        plsc.addupdate_scatter(st_v, [base], ones16, mask=gm)
        plsc.addupdate_scatter(st_v, [base + 1], d, mask=gm)
        plsc.addupdate_scatter(st_v, [base + 2], d * d, mask=gm)
        return carry
    lax.fori_loop(0, NPT // 16, sbody, 0)

    pltpu.sync_copy(st_v, st_sh.at[pl.ds(s * 512, 512)])
    plsc.subcore_barrier()

    @pl.when(s == 0)
    def _():
        pltpu.sync_copy(st_sh, stred_v)

        def rbody(j, carry):
            acc = stred_v[pl.ds(j * 16, 16)]
            for l in range(1, NS):
                acc = acc + stred_v[pl.ds(l * 512 + j * 16, 16)]
            st_v[pl.ds(j * 16, 16)] = acc
            return carry
        lax.fori_loop(0, 512 // 16, rbody, 0)
        pltpu.sync_copy(st_v, out_hbm.at[pl.ds(c * 512, 512)])


def _sc_stats(src, batch_pad):
    mesh = plsc.VectorSubcoreMesh(core_axis_name="c", subcore_axis_name="s",
                                  num_cores=NC, num_subcores=NS)
    f = functools.partial(
        pl.kernel,
        out_type=jax.ShapeDtypeStruct((NC * 512,), jnp.float32),
        mesh=mesh,
        scratch_types=[
            pltpu.VMEM((EPT,), jnp.int32),
            pltpu.VMEM((HALF,), jnp.float32),
            pltpu.VMEM((NS * NPT,), jnp.float32),
            pltpu.VMEM((NPT,), jnp.int32),
            pltpu.VMEM((512,), jnp.float32),
            pltpu.VMEM((NS * 512,), jnp.float32),
            pltpu.VMEM_SHARED((NS * HALF,), jnp.float32),
            pltpu.VMEM_SHARED((NS * 512,), jnp.float32),
        ],
        compiler_params=pltpu.CompilerParams(needs_layout_passes=False),
    )(_sc_stats_body)
    return f(src, batch_pad)


NB = 10
NBL = N_NODES // NB


def _pool_body(b_ref, e_ref, o_ref):
    i = pl.program_id(0)
    gid = lax.broadcasted_iota(jnp.int32, (B_GRAPHS, NBL), 0)
    oh = (b_ref[0] == gid).astype(jnp.float32)
    contrib = jnp.dot(oh, e_ref[...], preferred_element_type=jnp.float32,
                precision=lax.Precision.HIGHEST)

    @pl.when(i == 0)
    def _():
        o_ref[...] = contrib

    @pl.when(i != 0)
    def _():
        o_ref[...] = o_ref[...] + contrib


def _pool(batch3d, node_emb):
    return pl.pallas_call(
        _pool_body,
        grid=(NB,),
        in_specs=[
            pl.BlockSpec((1, 1, NBL), lambda i: (i, 0, 0)),
            pl.BlockSpec((NBL, CHANNELS), lambda i: (i, 0)),
        ],
        out_specs=pl.BlockSpec((B_GRAPHS, CHANNELS), lambda i: (0, 0)),
        out_shape=jax.ShapeDtypeStruct((B_GRAPHS, CHANNELS), jnp.float32),
    )(batch3d, node_emb)


def _head_body(stats_ref, pool_ref, wst_ref, bst_ref, wemb_ref, bemb_ref,
               wcomb_ref, bcomb_ref, wtok_ref, btok_ref, wlay_ref, blay_ref,
               wh1_ref, bh1_ref, wh2_ref, bh2_ref,
               tok_ref, lay_ref, head_ref):
    st = stats_ref[0] + stats_ref[1]
    Ncnt = st[:, 0:1]
    dsum = st[:, 1:2]
    dsq = st[:, 2:3]
    cnt = jnp.maximum(Ncnt, 1.0)
    Ecnt = 0.5 * dsum
    log_N = jnp.log(Ncnt + 1.0)
    log_E = jnp.log(Ecnt + 1.0)
    density = 2.0 * Ecnt / (Ncnt * (Ncnt - 1.0) + 1e-8)
    avg_deg = dsum / cnt
    deg_var = jnp.maximum(dsq / cnt - avg_deg * avg_deg, 0.0)

    feats = jnp.concatenate(
        [log_N, log_E, density, avg_deg, deg_var,
         jnp.zeros((B_GRAPHS, 3), jnp.float32)], axis=1)
    sh = jnp.dot(feats, wst_ref[...],
                 preferred_element_type=jnp.float32) + bst_ref[...]
    struct_h = jnp.maximum(sh, 0.0)

    pooled = pool_ref[...] / cnt
    emb_h = jnp.maximum(
        jnp.dot(pooled, wemb_ref[...], preferred_element_type=jnp.float32)
        + bemb_ref[...], 0.0)
    h = jnp.maximum(
        jnp.dot(struct_h + emb_h, wcomb_ref[...],
                preferred_element_type=jnp.float32) + bcomb_ref[...], 0.0)

    raw_tok = jax.nn.sigmoid(
        jnp.dot(h, wtok_ref[...], preferred_element_type=jnp.float32)
        + btok_ref[...])
    learned_tok = MIN_R + (MAX_R - MIN_R) * raw_tok

    m = jnp.mean(log_N)
    std = jnp.maximum(jnp.sqrt(jnp.mean((log_N - m) ** 2)), 1e-6)
    size_z = (log_N - m) / std
    prior_tok = MIN_R + (MAX_R - MIN_R) * jax.nn.sigmoid(-SIZE_TEMP * size_z)
    tok_ref[...] = (1.0 - SIZE_MIX) * learned_tok + SIZE_MIX * prior_tok

    lay_ref[...] = jax.nn.sigmoid(
        jnp.dot(h, wlay_ref[...], preferred_element_type=jnp.float32)
        + blay_ref[...])

    h1 = jnp.maximum(
        jnp.dot(h, wh1_ref[...], preferred_element_type=jnp.float32)
        + bh1_ref[...], 0.0)
    raw_head = jax.nn.sigmoid(
        jnp.dot(h1, wh2_ref[...], preferred_element_type=jnp.float32)
        + bh2_ref[...])
    learned_head = MIN_HG + (1.0 - MIN_HG) * raw_head
    head_prior = MIN_HG + (1.0 - MIN_HG) * jax.nn.sigmoid(-HEAD_TEMP * size_z)
    head_ref[...] = (1.0 - HEAD_MIX) * learned_head + HEAD_MIX * head_prior


def _head(stats3, poolsum, wst, bst, wemb, bemb, wcomb, bcomb,
          wtok, btok, wlay, blay, wh1, bh1, wh2, bh2):
    return pl.pallas_call(
        _head_body,
        out_shape=[
            jax.ShapeDtypeStruct((B_GRAPHS, 128), jnp.float32),
            jax.ShapeDtypeStruct((B_GRAPHS, 128), jnp.float32),
            jax.ShapeDtypeStruct((B_GRAPHS, 256), jnp.float32),
        ],
    )(stats3, poolsum, wst, bst, wemb, bemb, wcomb, bcomb,
      wtok, btok, wlay, blay, wh1, bh1, wh2, bh2)


def kernel(x, edge_index, batch, node_emb, W_struct, b_struct, W_emb, b_emb,
           W_comb, b_comb, W_tok, b_tok, W_layer, b_layer, W_h1, b_h1,
           W_h2, b_h2):
    src = edge_index[0]
    batch_pad = jnp.concatenate(
        [batch, jnp.zeros((N_PAD - N_NODES,), jnp.int32)])

    stats2 = _sc_stats(src, batch_pad)
    stats3 = stats2.reshape(NC, B_GRAPHS, 8)

    poolsum = _pool(batch.reshape(NB, 1, NBL), node_emb)

    wst = jnp.pad(W_struct.T, ((0, 3), (0, 0)))
    bst = b_struct.reshape(1, 64)
    wemb = W_emb.T
    bemb = b_emb.reshape(1, 64)
    wcomb = W_comb.T
    bcomb = b_comb.reshape(1, 64)
    wtok = jnp.pad(W_tok.T, ((0, 0), (0, 128 - NUM_LAYERS)))
    btok = jnp.pad(b_tok, (0, 128 - NUM_LAYERS)).reshape(1, 128)
    wlay = jnp.pad(W_layer.T, ((0, 0), (0, 128 - NUM_LAYERS)))
    blay = jnp.pad(b_layer, (0, 128 - NUM_LAYERS)).reshape(1, 128)
    wh1 = W_h1.T
    bh1 = b_h1.reshape(1, 64)
    nh = NUM_LAYERS * NUM_HEADS
    wh2 = jnp.pad(W_h2.T, ((0, 0), (0, 256 - nh)))
    bh2 = jnp.pad(b_h2, (0, 256 - nh)).reshape(1, 256)

    tok_pad, lay_pad, head_pad = _head(
        stats3, poolsum, wst, bst, wemb, bemb, wcomb, bcomb,
        wtok, btok, wlay, blay, wh1, bh1, wh2, bh2)

    token_ratios = tok_pad[:, :NUM_LAYERS]
    layer_gates = lay_pad[:, :NUM_LAYERS]
    head_gates = head_pad[:, :nh].reshape(B_GRAPHS, NUM_LAYERS, NUM_HEADS)
    return (token_ratios, layer_gates, head_gates)


# speedup vs baseline: 54.2289x; 1.2579x over previous
import functools

import jax
import jax.numpy as jnp
from jax import lax
from jax.experimental import pallas as pl
from jax.experimental.pallas import tpu as pltpu
from jax.experimental.pallas import tpu_sc as plsc

N_NODES = 10000
N_EDGES = 320000
B_GRAPHS = 64
CHANNELS = 128

NC = 2
NS = 16
HALF = 5120
N_PAD = NC * HALF
NPT = HALF // NS
EPT = N_EDGES // NS

MIN_R = 0.2
MAX_R = 1.0
MIN_HG = 0.05
SIZE_MIX = 0.3
SIZE_TEMP = 3.0
HEAD_MIX = 0.3
HEAD_TEMP = 2.0
NUM_LAYERS = 12
NUM_HEADS = 12


def _sc_stats_body(edge_hbm, batch_hbm, out_hbm,
                   e_v, hist_v, part_v, b_v, st_v, stred_v,
                   hist_sh, st_sh):
    c = lax.axis_index("c")
    s = lax.axis_index("s")
    lo = c * HALF

    zf = jnp.zeros((16,), jnp.float32)
    ones16 = jnp.ones((16,), jnp.float32)

    def zbody(i, carry):
        hist_v[pl.ds(i * 16, 16)] = zf
        return carry
    lax.fori_loop(0, HALF // 16, zbody, 0, unroll=4)

    pltpu.sync_copy(edge_hbm.at[pl.ds(s * EPT, EPT)], e_v)

    def hbody(i, carry):
        idx = e_v[pl.ds(i * 16, 16)] - lo
        m = (idx >= 0) & (idx < HALF)
        plsc.addupdate_scatter(hist_v, [idx], ones16, mask=m)
        return carry
    lax.fori_loop(0, EPT // 16, hbody, 0, unroll=4)

    pltpu.sync_copy(hist_v, hist_sh.at[pl.ds(s * HALF, HALF)])
    plsc.subcore_barrier()

    nbase0 = lo + s * NPT
    loff0 = jnp.minimum(nbase0, N_NODES - NPT) - lo
    for l in range(NS):
        pltpu.sync_copy(hist_sh.at[pl.ds(l * HALF + loff0, NPT)],
                        part_v.at[pl.ds(l * NPT, NPT)])

    def zsbody(i, carry):
        st_v[pl.ds(i * 16, 16)] = zf
        return carry
    lax.fori_loop(0, 512 // 16, zsbody, 0)

    nbase = lo + s * NPT
    o = jnp.minimum(nbase, N_NODES - NPT)
    loff = o - lo
    pltpu.sync_copy(batch_hbm.at[pl.ds(o, NPT)], b_v)

    lane = lax.iota(jnp.int32, 16)

    def sbody(j, carry):
        d = part_v[pl.ds(j * 16, 16)]
        for l in range(1, NS):
            d = d + part_v[pl.ds(l * NPT + j * 16, 16)]
        g = b_v[pl.ds(j * 16, 16)]
        gid = o + j * 16 + lane
        gm = (gid >= nbase) & (gid < N_NODES)
        base = g * 8
        plsc.addupdate_scatter(st_v, [base], ones16, mask=gm)
        plsc.addupdate_scatter(st_v, [base + 1], d, mask=gm)
        plsc.addupdate_scatter(st_v, [base + 2], d * d, mask=gm)
        return carry
    lax.fori_loop(0, NPT // 16, sbody, 0)

    pltpu.sync_copy(st_v, st_sh.at[pl.ds(s * 512, 512)])
    plsc.subcore_barrier()

    @pl.when(s == 0)
    def _():
        pltpu.sync_copy(st_sh, stred_v)

        def rbody(j, carry):
            acc = stred_v[pl.ds(j * 16, 16)]
            for l in range(1, NS):
                acc = acc + stred_v[pl.ds(l * 512 + j * 16, 16)]
            st_v[pl.ds(j * 16, 16)] = acc
            return carry
        lax.fori_loop(0, 512 // 16, rbody, 0)
        pltpu.sync_copy(st_v, out_hbm.at[pl.ds(c * 512, 512)])


def _extract_body(e_ref, o_ref):
    o_ref[...] = e_ref[0, :]


def _extract_src(edge_index):
    return pl.pallas_call(
        _extract_body,
        out_shape=jax.ShapeDtypeStruct((N_EDGES,), jnp.int32),
    )(edge_index)


def _sc_stats(src, batch):
    mesh = plsc.VectorSubcoreMesh(core_axis_name="c", subcore_axis_name="s",
                                  num_cores=NC, num_subcores=NS)
    f = functools.partial(
        pl.kernel,
        out_type=jax.ShapeDtypeStruct((NC * 512,), jnp.float32),
        mesh=mesh,
        scratch_types=[
            pltpu.VMEM((EPT,), jnp.int32),
            pltpu.VMEM((HALF,), jnp.float32),
            pltpu.VMEM((NS * NPT,), jnp.float32),
            pltpu.VMEM((NPT,), jnp.int32),
            pltpu.VMEM((512,), jnp.float32),
            pltpu.VMEM((NS * 512,), jnp.float32),
            pltpu.VMEM_SHARED((NS * HALF,), jnp.float32),
            pltpu.VMEM_SHARED((NS * 512,), jnp.float32),
        ],
        compiler_params=pltpu.CompilerParams(needs_layout_passes=False),
    )(_sc_stats_body)
    return f(src, batch)


NB = 10
NBL = N_NODES // NB


def _pool_body(b_ref, e_ref, o_ref):
    i = pl.program_id(0)
    gid = lax.broadcasted_iota(jnp.int32, (B_GRAPHS, NBL), 0)
    oh = (b_ref[0] == gid).astype(jnp.float32)
    contrib = jnp.dot(oh, e_ref[...], preferred_element_type=jnp.float32,
                precision=lax.Precision.HIGHEST)

    @pl.when(i == 0)
    def _():
        o_ref[...] = contrib

    @pl.when(i != 0)
    def _():
        o_ref[...] = o_ref[...] + contrib


def _pool(batch3d, node_emb):
    return pl.pallas_call(
        _pool_body,
        grid=(NB,),
        in_specs=[
            pl.BlockSpec((1, 1, NBL), lambda i: (i, 0, 0)),
            pl.BlockSpec((NBL, CHANNELS), lambda i: (i, 0)),
        ],
        out_specs=pl.BlockSpec((B_GRAPHS, CHANNELS), lambda i: (0, 0)),
        out_shape=jax.ShapeDtypeStruct((B_GRAPHS, CHANNELS), jnp.float32),
    )(batch3d, node_emb)


def _head_body(stats_ref, pool_ref, wst_ref, bst_ref, wemb_ref, bemb_ref,
               wcomb_ref, bcomb_ref, wtok_ref, btok_ref, wlay_ref, blay_ref,
               wh1_ref, bh1_ref, wh2_ref, bh2_ref,
               tok_ref, lay_ref, head_ref):
    def nt_dot(a, w_ref):
        return lax.dot_general(a, w_ref[...], (((1,), (1,)), ((), ())),
                               preferred_element_type=jnp.float32)

    st = stats_ref[0] + stats_ref[1]
    Ncnt = st[:, 0:1]
    dsum = st[:, 1:2]
    dsq = st[:, 2:3]
    cnt = jnp.maximum(Ncnt, 1.0)
    Ecnt = 0.5 * dsum
    log_N = jnp.log(Ncnt + 1.0)
    log_E = jnp.log(Ecnt + 1.0)
    density = 2.0 * Ecnt / (Ncnt * (Ncnt - 1.0) + 1e-8)
    avg_deg = dsum / cnt
    deg_var = jnp.maximum(dsq / cnt - avg_deg * avg_deg, 0.0)

    feats = jnp.concatenate(
        [log_N, log_E, density, avg_deg, deg_var], axis=1)
    struct_h = jnp.maximum(nt_dot(feats, wst_ref) + bst_ref[...], 0.0)

    pooled = pool_ref[...] / cnt
    emb_h = jnp.maximum(nt_dot(pooled, wemb_ref) + bemb_ref[...], 0.0)
    h = jnp.maximum(nt_dot(struct_h + emb_h, wcomb_ref) + bcomb_ref[...], 0.0)

    raw_tok = jax.nn.sigmoid(nt_dot(h, wtok_ref) + btok_ref[...])
    learned_tok = MIN_R + (MAX_R - MIN_R) * raw_tok

    m = jnp.mean(log_N)
    std = jnp.maximum(jnp.sqrt(jnp.mean((log_N - m) ** 2)), 1e-6)
    size_z = (log_N - m) / std
    prior_tok = MIN_R + (MAX_R - MIN_R) * jax.nn.sigmoid(-SIZE_TEMP * size_z)
    tok_ref[...] = (1.0 - SIZE_MIX) * learned_tok + SIZE_MIX * prior_tok

    lay_ref[...] = jax.nn.sigmoid(nt_dot(h, wlay_ref) + blay_ref[...])

    h1 = jnp.maximum(nt_dot(h, wh1_ref) + bh1_ref[...], 0.0)
    raw_head = jax.nn.sigmoid(nt_dot(h1, wh2_ref) + bh2_ref[...])
    learned_head = MIN_HG + (1.0 - MIN_HG) * raw_head
    head_prior = MIN_HG + (1.0 - MIN_HG) * jax.nn.sigmoid(-HEAD_TEMP * size_z)
    head_ref[...] = (1.0 - HEAD_MIX) * learned_head + HEAD_MIX * head_prior


def _head(stats3, poolsum, wst, bst, wemb, bemb, wcomb, bcomb,
          wtok, btok, wlay, blay, wh1, bh1, wh2, bh2):
    nh = NUM_LAYERS * NUM_HEADS
    return pl.pallas_call(
        _head_body,
        out_shape=[
            jax.ShapeDtypeStruct((B_GRAPHS, NUM_LAYERS), jnp.float32),
            jax.ShapeDtypeStruct((B_GRAPHS, NUM_LAYERS), jnp.float32),
            jax.ShapeDtypeStruct((B_GRAPHS, nh), jnp.float32),
        ],
    )(stats3, poolsum, wst, bst.reshape(1, -1), wemb, bemb.reshape(1, -1),
      wcomb, bcomb.reshape(1, -1), wtok, btok.reshape(1, -1),
      wlay, blay.reshape(1, -1), wh1, bh1.reshape(1, -1),
      wh2, bh2.reshape(1, -1))


def kernel(x, edge_index, batch, node_emb, W_struct, b_struct, W_emb, b_emb,
           W_comb, b_comb, W_tok, b_tok, W_layer, b_layer, W_h1, b_h1,
           W_h2, b_h2):
    src = _extract_src(edge_index)
    stats2 = _sc_stats(src, batch)
    stats3 = stats2.reshape(NC, B_GRAPHS, 8)

    poolsum = _pool(batch.reshape(NB, 1, NBL), node_emb)

    token_ratios, layer_gates, head_flat = _head(
        stats3, poolsum, W_struct, b_struct, W_emb, b_emb, W_comb, b_comb,
        W_tok, b_tok, W_layer, b_layer, W_h1, b_h1, W_h2, b_h2)
    head_gates = head_flat.reshape(B_GRAPHS, NUM_LAYERS, NUM_HEADS)
    return (token_ratios, layer_gates, head_gates)
